# baseline (speedup 1.0000x reference)
import jax
import jax.numpy as jnp
from jax import lax
from jax.experimental import pallas as pl
from jax.experimental.pallas import tpu as pltpu
from jax.experimental.pallas import tpu_sc as plsc

N_MID = 1000000
N_CAT = 100000
EMBED_DIM = 32
BATCH = 16384
MAX_LEN = 200

NW = 32
BW = BATCH // NW
D2 = 2 * EMBED_DIM
DBLK = D2 // 8
BBLK = BW // 128


def _body(mid_idxT, cat_idxT, mid_table, cat_table, out_hbm,
          midx_v, cidx_v, mrows_v, crows_v, tbuf_v, gsem, wsem):
    wid = lax.axis_index("c") * 16 + lax.axis_index("s")
    b0 = wid * BW
    bb0 = wid * BBLK
    lane = lax.iota(jnp.int32, 16)

    def chunk(l, _):
        pltpu.sync_copy(mid_idxT.at[l, pl.ds(b0, BW)], midx_v)
        pltpu.sync_copy(cat_idxT.at[l, pl.ds(b0, BW)], cidx_v)
        gathers = []
        for s in range(BW // 128):
            cm = pltpu.make_async_copy(
                mid_table.at[midx_v.at[pl.ds(s * 128, 128)]],
                mrows_v.at[pl.ds(s * 128, 128), :], gsem)
            cc = pltpu.make_async_copy(
                cat_table.at[cidx_v.at[pl.ds(s * 128, 128)]],
                crows_v.at[pl.ds(s * 128, 128), :], gsem)
            cm.start()
            cc.start()
            gathers.append(cm)
            gathers.append(cc)
        for c in gathers:
            c.wait()

        def transpose_group(g, _):
            rows = g * 16 + lane
            for d in range(EMBED_DIM):
                col = jnp.full((16,), d, jnp.int32)
                tbuf_v[d, pl.ds(g * 16, 16)] = plsc.load_gather(
                    mrows_v, [rows, col])
                tbuf_v[EMBED_DIM + d, pl.ds(g * 16, 16)] = plsc.load_gather(
                    crows_v, [rows, col])
            return ()

        lax.fori_loop(0, BW // 16, transpose_group, (), unroll=8)

        writes = []
        for db in range(DBLK):
            for bb in range(BBLK):
                w = pltpu.make_async_copy(
                    tbuf_v.at[pl.ds(db * 8, 8), pl.ds(bb * 128, 128)],
                    out_hbm.at[l, db, bb0 + bb, :, :], wsem)
                w.start()
                writes.append(w)
        for c in writes:
            c.wait()
        return ()

    lax.fori_loop(0, MAX_LEN, chunk, (), unroll=False)


@jax.jit
def _run(mid_idxT, cat_idxT, mid_table, cat_table):
    mesh = plsc.VectorSubcoreMesh(core_axis_name="c", subcore_axis_name="s")
    f = pl.kernel(
        _body,
        out_type=jax.ShapeDtypeStruct(
            (MAX_LEN, DBLK, BATCH // 128, 8, 128), jnp.float32),
        mesh=mesh,
        scratch_types=[
            pltpu.VMEM((BW,), jnp.int32),
            pltpu.VMEM((BW,), jnp.int32),
            pltpu.VMEM((BW, EMBED_DIM), jnp.float32),
            pltpu.VMEM((BW, EMBED_DIM), jnp.float32),
            pltpu.VMEM((D2, BW), jnp.float32),
            pltpu.SemaphoreType.DMA,
            pltpu.SemaphoreType.DMA,
        ],
        compiler_params=pltpu.CompilerParams(use_tc_tiling_on_sc=False,
                                             needs_layout_passes=False,
                                             disable_bounds_checks=True),
    )
    return f(mid_idxT, cat_idxT, mid_table, cat_table)


def kernel(mid_his_input, cat_his_input, mid_table, cat_table):
    mid_idxT = mid_his_input.astype(jnp.int32).T
    cat_idxT = cat_his_input.astype(jnp.int32).T
    out5 = _run(mid_idxT, cat_idxT, mid_table, cat_table)
    t = jnp.transpose(out5, (2, 4, 0, 1, 3))
    return t.reshape(BATCH, MAX_LEN, D2)


# speedup vs baseline: 1.9844x; 1.9844x over previous
import jax
import jax.numpy as jnp
from jax import lax
from jax.experimental import pallas as pl
from jax.experimental.pallas import tpu as pltpu
from jax.experimental.pallas import tpu_sc as plsc

N_MID = 1000000
N_CAT = 100000
EMBED_DIM = 32
BATCH = 16384
MAX_LEN = 200

NW = 32
BPC = 4
BATCH_PER_W = BATCH // NW
CHUNKS_PER_W = BATCH_PER_W // BPC
GATHER_SPLITS = ((0, 128), (128, 72))


def _body(mid_idx_hbm, cat_idx_hbm, mid_table, cat_table, out_hbm,
          midx_v, cidx_v, mrows_v, crows_v, gsem0, gsem1, wsem0, wsem1):
    wid = lax.axis_index("c") * 16 + lax.axis_index("s")
    b0 = wid * BATCH_PER_W
    gsems = (gsem0, gsem1)
    wsems = (wsem0, wsem1)

    def gather_copies(buf):
        copies = []
        for i in range(BPC):
            for (l0, n) in GATHER_SPLITS:
                copies.append(pltpu.make_async_copy(
                    mid_table.at[midx_v.at[buf, i, pl.ds(l0, n)]],
                    mrows_v.at[buf, i, pl.ds(l0, n), :], gsems[buf]))
                copies.append(pltpu.make_async_copy(
                    cat_table.at[cidx_v.at[buf, i, pl.ds(l0, n)]],
                    crows_v.at[buf, i, pl.ds(l0, n), :], gsems[buf]))
        return copies

    def write_copies(t, buf):
        b = b0 + t * BPC
        return (
            pltpu.make_async_copy(
                mrows_v.at[buf],
                out_hbm.at[pl.ds(b, BPC), :, pl.ds(0, EMBED_DIM)],
                wsems[buf]),
            pltpu.make_async_copy(
                crows_v.at[buf],
                out_hbm.at[pl.ds(b, BPC), :, pl.ds(EMBED_DIM, EMBED_DIM)],
                wsems[buf]),
        )

    def fire(t, buf):
        b = b0 + t * BPC
        pltpu.sync_copy(mid_idx_hbm.at[pl.ds(b, BPC)], midx_v.at[buf])
        pltpu.sync_copy(cat_idx_hbm.at[pl.ds(b, BPC)], cidx_v.at[buf])
        for c in gather_copies(buf):
            c.start()

    def drain_write(t, buf):
        for c in gather_copies(buf):
            c.wait()
        for c in write_copies(t, buf):
            c.start()

    def wait_writes(t, buf):
        for c in write_copies(t, buf):
            c.wait()

    fire(0, 0)

    def pair(t2, _):
        t = t2 * 2
        fire(t + 1, 1)
        drain_write(t, 0)
        wait_writes(t, 0)
        @pl.when(t + 2 < CHUNKS_PER_W)
        def _():
            fire(t + 2, 0)
        drain_write(t + 1, 1)
        wait_writes(t + 1, 1)
        return ()

    lax.fori_loop(0, CHUNKS_PER_W // 2, pair, (), unroll=False)


@jax.jit
def _run(mid_idx, cat_idx, mid_table, cat_table):
    mesh = plsc.VectorSubcoreMesh(core_axis_name="c", subcore_axis_name="s")
    f = pl.kernel(
        _body,
        out_type=jax.ShapeDtypeStruct((BATCH, MAX_LEN, 2 * EMBED_DIM),
                                      jnp.float32),
        mesh=mesh,
        scratch_types=[
            pltpu.VMEM((2, BPC, MAX_LEN), jnp.int32),
            pltpu.VMEM((2, BPC, MAX_LEN), jnp.int32),
            pltpu.VMEM((2, BPC, MAX_LEN, EMBED_DIM), jnp.float32),
            pltpu.VMEM((2, BPC, MAX_LEN, EMBED_DIM), jnp.float32),
            pltpu.SemaphoreType.DMA,
            pltpu.SemaphoreType.DMA,
            pltpu.SemaphoreType.DMA,
            pltpu.SemaphoreType.DMA,
        ],
        compiler_params=pltpu.CompilerParams(use_tc_tiling_on_sc=False),
    )
    return f(mid_idx, cat_idx, mid_table, cat_table)


def kernel(mid_his_input, cat_his_input, mid_table, cat_table):
    mid_idx = mid_his_input.astype(jnp.int32)
    cat_idx = cat_his_input.astype(jnp.int32)
    return _run(mid_idx, cat_idx, mid_table, cat_table)
